# Initial kernel scaffold; baseline (speedup 1.0000x reference)
import jax
import jax.numpy as jnp
from jax.experimental import pallas as pl


def kernel(query, keys, VT, k, ef_search):
    raise NotImplementedError("write your pallas kernel here")


# speedup vs baseline: 7.7101x; 7.7101x over previous
import functools

import jax
import jax.numpy as jnp
from jax import lax
from jax.experimental import pallas as pl
from jax.experimental.pallas import tpu as pltpu
from jax.experimental.pallas import tpu_sc as plsc

D = 128
DP = 64
N = 100000
NPAD = 100352
KB = 2048
NKB = NPAD // KB
QB = 128
NQB = 8
NQ = 1024
CH = 128
NCH = NPAD // CH
CHB = KB // CH
NCHP = 896
CSEG = NCHP // 128
CQ = 64
CAP_CH = 144
ESEG = CAP_CH
EQ = 16
E1W = 2304
E2Q = 48
E2W = 1024
EF = 128
TK = 10
BIG = 3.0e38
QPW = NQ // 32


def _cumsum_lanes(m, width):
    iota = lax.broadcasted_iota(jnp.int32, m.shape, 1)
    acc = m
    k = 1
    while k < width:
        acc = acc + jnp.where(iota >= k, pltpu.roll(acc, k, axis=1), 0)
        k *= 2
    return acc


def _bitonic(d, i, width):
    li = lax.broadcasted_iota(jnp.int32, d.shape, 1)
    k = 2
    while k <= width:
        j = k // 2
        while j >= 1:
            lo = (li & j) == 0
            w = d.shape[1]
            pd = jnp.where(lo, pltpu.roll(d, w - j, axis=1), pltpu.roll(d, j, axis=1))
            pi = jnp.where(lo, pltpu.roll(i, w - j, axis=1), pltpu.roll(i, j, axis=1))
            less = (d < pd) | ((d == pd) & (i < pi))
            take_min = ((li & k) == 0) == lo
            d = jnp.where(take_min == less, d, pd)
            i = jnp.where(take_min == less, i, pi)
            j //= 2
        k *= 2
    return d, i


def _quota(vals, ids, rank, tot, quota):
    cols_v, cols_i = [], []
    posl = []
    for s in range(quota):
        posl.append(jnp.sum((rank <= s).astype(jnp.int32), axis=1, keepdims=True))
    pos = jnp.concatenate(posl, axis=1)
    sio = lax.broadcasted_iota(jnp.int32, pos.shape, 1)
    valid = sio < tot
    posc = jnp.minimum(pos, vals.shape[1] - 1)
    cv = jnp.take_along_axis(vals, posc, axis=1)
    gv = jnp.take_along_axis(ids, posc, axis=1)
    return jnp.where(valid, cv, BIG), jnp.where(valid, gv, NCH - 1)


def _rot_body(a_ref, vt_ref, o_ref):
    o_ref[...] = lax.dot_general(
        a_ref[...], vt_ref[...], (((1,), (1,)), ((), ())),
        preferred_element_type=jnp.float32)


def _rotate(a, VT, blk):
    n = a.shape[0]
    return pl.pallas_call(
        _rot_body,
        grid=(n // blk,),
        in_specs=[pl.BlockSpec((blk, D), lambda i: (i, 0)),
                  pl.BlockSpec((D, D), lambda i: (0, 0))],
        out_specs=pl.BlockSpec((blk, D), lambda i: (i, 0)),
        out_shape=jax.ShapeDtypeStruct((n, D), jnp.float32),
    )(a, VT)


def _norm_body(a_ref, o_ref):
    a = a_ref[:, :DP]
    o_ref[...] = jnp.sum(a * a, axis=1, keepdims=True)


def _norms(a, blk):
    n = a.shape[0]
    return pl.pallas_call(
        _norm_body,
        grid=(n // blk,),
        in_specs=[pl.BlockSpec((blk, D), lambda i: (i, 0))],
        out_specs=pl.BlockSpec((blk, 1), lambda i: (i, 0)),
        out_shape=jax.ShapeDtypeStruct((n, 1), jnp.float32),
    )(a)


def _coarse_body(q_ref, x_ref, qn_ref, xn_ref, d_ref, cm_ref):
    qp = q_ref[:, :DP]
    xp = x_ref[:, :DP]
    s = lax.dot_general(qp, xp, (((1,), (1,)), ((), ())),
                        preferred_element_type=jnp.float32)
    d = (qn_ref[...] - 2.0 * s) + xn_ref[...]
    d_ref[...] = d
    cm_ref[...] = jnp.min(d.reshape(QB, CHB, CH), axis=2).reshape(1, QB, CHB)


def _coarse(q, x, qn, xn):
    return pl.pallas_call(
        _coarse_body,
        grid=(NQB, NKB),
        in_specs=[pl.BlockSpec((QB, D), lambda qi, ki: (qi, 0)),
                  pl.BlockSpec((KB, D), lambda qi, ki: (ki, 0)),
                  pl.BlockSpec((QB, 1), lambda qi, ki: (qi, 0)),
                  pl.BlockSpec((1, KB), lambda qi, ki: (0, ki))],
        out_specs=[pl.BlockSpec((QB, KB), lambda qi, ki: (qi, ki)),
                   pl.BlockSpec((1, QB, CHB), lambda qi, ki: (ki, qi, 0))],
        out_shape=[jax.ShapeDtypeStruct((NQ, NPAD), jnp.float32),
                   jax.ShapeDtypeStruct((NKB, NQ, CHB), jnp.float32)],
    )(q, x, qn, xn)


def _thresh_body(cm_ref, t_ref):
    cm = cm_ref[...]

    def it(_, lh):
        lo, hi = lh
        mid = 0.5 * (lo + hi)
        cnt = jnp.sum((cm <= mid).astype(jnp.int32), axis=1, keepdims=True)
        sel = cnt >= 129
        return (jnp.where(sel, lo, mid), jnp.where(sel, mid, hi))

    lo0 = jnp.full((NQ, 1), -1.0e6, jnp.float32)
    hi0 = jnp.full((NQ, 1), 1.0e6, jnp.float32)
    lo, hi = lax.fori_loop(0, 48, it, (lo0, hi0))
    t_ref[...] = hi


def _thresh(cmt):
    return pl.pallas_call(
        _thresh_body,
        out_shape=jax.ShapeDtypeStruct((NQ, 1), jnp.float32),
    )(cmt)


def _chsel_body(cm_ref, t_ref, sci_ref):
    cm = cm_ref[...]
    t = t_ref[...]
    pad = jnp.full((QB, NCHP - NCH), BIG, jnp.float32)
    cmp_ = jnp.concatenate([cm, pad], axis=1)
    svl, sil = [], []
    for s in range(CSEG):
        vals = cmp_[:, s * 128:(s + 1) * 128]
        cid = s * 128 + lax.broadcasted_iota(jnp.int32, (QB, 128), 1)
        m = vals <= t
        rank = _cumsum_lanes(m.astype(jnp.int32), 128)
        tot = rank[:, 127:128]
        cv, gv = _quota(jnp.where(m, vals, BIG), cid, rank, tot, CQ)
        svl.append(cv)
        sil.append(gv)
    if 512 > CSEG * CQ:
        svl.append(jnp.full((QB, 512 - CSEG * CQ), BIG, jnp.float32))
        sil.append(jnp.full((QB, 512 - CSEG * CQ), NCH - 1, jnp.int32))
    sv = jnp.concatenate(svl, axis=1)
    si = jnp.concatenate(sil, axis=1)
    key = jnp.where(sv < BIG, si, jnp.int32(1000000000))
    _, si = _bitonic(key, si, 512)
    sci_ref[...] = si[:, :256]


def _chsel(cmt, t):
    return pl.pallas_call(
        _chsel_body,
        grid=(NQB,),
        in_specs=[pl.BlockSpec((QB, NCH), lambda qi: (qi, 0)),
                  pl.BlockSpec((QB, 1), lambda qi: (qi, 0))],
        out_specs=pl.BlockSpec((QB, 256), lambda qi: (qi, 0)),
        out_shape=jax.ShapeDtypeStruct((NQ, 256), jnp.int32),
    )(cmt, t)


def _gatherd_sc(dflat, ri):
    mesh = plsc.VectorSubcoreMesh(core_axis_name="c", subcore_axis_name="s")

    @functools.partial(
        pl.kernel, mesh=mesh,
        out_type=jax.ShapeDtypeStruct((NQ, CAP_CH, CH), jnp.float32),
        scratch_types=[
            pltpu.VMEM((CAP_CH,), jnp.int32),
            pltpu.VMEM((CAP_CH, CH), jnp.float32),
            pltpu.SemaphoreType.DMA,
        ],
    )
    def k(d_h, ri_h, gd_h, idx_v, rows_v, sem):
        wid = lax.axis_index("s") * 2 + lax.axis_index("c")
        qbase = wid * QPW

        def qloop(ql, _):
            q = qbase + ql
            pltpu.sync_copy(ri_h.at[q], idx_v)
            pltpu.async_copy(d_h.at[idx_v], rows_v, sem).wait()
            pltpu.sync_copy(rows_v, gd_h.at[q])
            return 0

        lax.fori_loop(0, QPW, qloop, 0)

    return k(dflat, ri)


def _compact_body(gd_ref, t_ref, ci_ref, s1v_ref, s1i_ref):
    t = t_ref[...]
    lane = lax.broadcasted_iota(jnp.int32, (QB, 128), 1)

    def grp(g, _):
        cvl, gvl = [], []
        for u in range(8):
            s = g * 8 + u
            vals = gd_ref[:, pl.ds(pl.multiple_of(s * 128, 128), 128)]
            gidx = s * 128 + lane
            m = vals <= t
            rank = _cumsum_lanes(m.astype(jnp.int32), 128)
            tot = rank[:, 127:128]
            cv, gv = _quota(jnp.where(m, vals, BIG), gidx, rank, tot, EQ)
            cvl.append(cv)
            gvl.append(gv)
        s1v_ref[:, pl.ds(pl.multiple_of(g * 128, 128), 128)] = jnp.concatenate(cvl, axis=1)
        s1i_ref[:, pl.ds(pl.multiple_of(g * 128, 128), 128)] = jnp.concatenate(gvl, axis=1)
        return 0

    lax.fori_loop(0, ESEG // 8, grp, 0)
    s2vl, s2il = [], []
    for s in range(E1W // 128):
        vals = s1v_ref[:, s * 128:(s + 1) * 128]
        ids = s1i_ref[:, s * 128:(s + 1) * 128]
        m = vals < BIG
        rank = _cumsum_lanes(m.astype(jnp.int32), 128)
        tot = rank[:, 127:128]
        cv, gv = _quota(vals, ids, rank, tot, E2Q)
        s2vl.append(cv)
        s2il.append(gv)
    if E2W > (E1W // 128) * E2Q:
        s2vl.append(jnp.full((QB, E2W - (E1W // 128) * E2Q), BIG, jnp.float32))
        s2il.append(jnp.full((QB, E2W - (E1W // 128) * E2Q), 0, jnp.int32))
    s2v = jnp.concatenate(s2vl, axis=1)
    s2i = jnp.concatenate(s2il, axis=1)
    s2v, s2i = _bitonic(s2v, s2i, E2W)
    ci_ref[...] = s2i[:, :EF]


def _compact(gd, t):
    return pl.pallas_call(
        _compact_body,
        grid=(NQB,),
        in_specs=[pl.BlockSpec((QB, CAP_CH * CH), lambda qi: (qi, 0)),
                  pl.BlockSpec((QB, 1), lambda qi: (qi, 0))],
        out_specs=pl.BlockSpec((QB, EF), lambda qi: (qi, 0)),
        out_shape=jax.ShapeDtypeStruct((NQ, EF), jnp.int32),
        scratch_shapes=[pltpu.VMEM((QB, E1W), jnp.float32),
                        pltpu.VMEM((QB, E1W), jnp.int32)],
    )(gd, t)


def _gatherx_sc(x_pad, ci_flat):
    mesh = plsc.VectorSubcoreMesh(core_axis_name="c", subcore_axis_name="s")
    rows_total = NQ * EF
    rpw = rows_total // 32
    blk = 512

    @functools.partial(
        pl.kernel, mesh=mesh,
        out_type=jax.ShapeDtypeStruct((rows_total, D), jnp.float32),
        scratch_types=[
            pltpu.VMEM((blk,), jnp.int32),
            pltpu.VMEM((blk, D), jnp.float32),
            pltpu.SemaphoreType.DMA,
        ],
    )
    def k(x_h, ci_h, g_h, idx_v, rows_v, sem):
        wid = lax.axis_index("s") * 2 + lax.axis_index("c")
        base = wid * rpw

        def chunk(i, _):
            off = base + i * blk
            pltpu.sync_copy(ci_h.at[pl.ds(off, blk)], idx_v)
            pltpu.async_copy(x_h.at[idx_v], rows_v, sem).wait()
            pltpu.sync_copy(rows_v, g_h.at[pl.ds(off, blk)])
            return 0

        lax.fori_loop(0, rpw // blk, chunk, 0)

    return k(x_pad, ci_flat)


def _refine_body(q_ref, g_ref, ci_ref, oi_ref, od_ref):
    qv3 = q_ref[...].reshape(QB, 1, D)
    g = g_ref[...]
    diff = qv3 - g
    df = jnp.sum(diff * diff, axis=2)
    slot = lax.broadcasted_iota(jnp.int32, (QB, EF), 1)
    sd, ss = _bitonic(df, slot, EF)
    oi = jnp.take_along_axis(ci_ref[...], ss, axis=1)
    oi_ref[...] = oi[:, :TK]
    od_ref[...] = sd[:, :TK]


def _refine(q, g3, ci):
    return pl.pallas_call(
        _refine_body,
        grid=(NQB,),
        in_specs=[pl.BlockSpec((QB, D), lambda qi: (qi, 0)),
                  pl.BlockSpec((QB, EF, D), lambda qi: (qi, 0, 0)),
                  pl.BlockSpec((QB, EF), lambda qi: (qi, 0))],
        out_specs=[pl.BlockSpec((QB, TK), lambda qi: (qi, 0)),
                   pl.BlockSpec((QB, TK), lambda qi: (qi, 0))],
        out_shape=[jax.ShapeDtypeStruct((NQ, TK), jnp.int32),
                   jax.ShapeDtypeStruct((NQ, TK), jnp.float32)],
    )(q, g3, ci)


def kernel(query, keys, VT, k, ef_search):
    q = _rotate(query, VT, 1024)
    x = _rotate(keys, VT, 2000)
    x_pad = jnp.pad(x, ((0, NPAD - N), (0, 0)))
    qn = _norms(q, 1024)
    xn = _norms(x_pad, KB).T
    pad_mask = jnp.arange(NPAD)[None, :] >= N
    xn_pad = jnp.where(pad_mask, BIG, xn)
    d, cm3 = _coarse(q, x_pad, qn, xn_pad)
    cmt = cm3.transpose(1, 0, 2).reshape(NQ, NCH)
    t = _thresh(cmt)
    sci = _chsel(cmt, t)
    ri = sci[:, :CAP_CH] + jnp.arange(NQ, dtype=jnp.int32)[:, None] * NCH
    gd = _gatherd_sc(d.reshape(NQ * NCH, CH), ri)
    ci_pos = _compact(gd.reshape(NQ, CAP_CH * CH), t)
    cid = jnp.take_along_axis(sci[:, :CAP_CH], ci_pos // CH, axis=1)
    ci = cid * CH + ci_pos % CH
    g = _gatherx_sc(x_pad, ci.reshape(NQ * EF))
    oi, od = _refine(q, g.reshape(NQ, EF, D), ci)
    zero_dep = (jnp.asarray(k) * 0 + jnp.asarray(ef_search) * 0).astype(oi.dtype)
    return oi + zero_dep, od
